# baseline (speedup 1.0000x reference)
import functools

import jax
import jax.numpy as jnp
from jax import lax
from jax.experimental import pallas as pl
from jax.experimental.pallas import tpu as pltpu
from jax.experimental.pallas import tpu_sc as plsc

N = 10000
E = 320000
D = 128
DH = 256
EPS = 1e-7
BN_EPS = 1e-5

NC = 2
NS = 16
CH = 88
NK = 232
GC = 8
NG = NK // GC
EP = NS * NK * CH
RPT = 624


def _edge_sums_body(table_hbm, edges_hbm, out_hbm,
                    idx_v, rows_v, acc, semg0, semg1, semi):
    c = lax.axis_index("c")
    s = lax.axis_index("s")
    cN = (c * N).astype(jnp.int32)
    buf0 = rows_v.at[0]
    buf1 = rows_v.at[1]

    zvec = jnp.zeros((16,), jnp.float32)

    def zero_row(i, carry):
        for j in range(D // 16):
            rows_v[0, i, pl.ds(j * 16, 16)] = zvec
        return carry

    lax.fori_loop(0, CH, zero_row, 0)
    base = s * RPT
    for r in range(RPT // CH):
        pltpu.sync_copy(buf0, acc.at[pl.ds(base + r * CH, CH)])
    rem = RPT - (RPT // CH) * CH
    pltpu.sync_copy(buf0.at[pl.ds(0, rem)],
                    acc.at[pl.ds(base + (RPT // CH) * CH, rem)])

    @pl.when(s == 0)
    def _():
        pltpu.sync_copy(buf0.at[pl.ds(0, 16)], acc.at[pl.ds(NS * RPT, 16)])

    plsc.subcore_barrier()

    bufs = (buf0, buf1)
    gsems = (semg0, semg1)

    def gather(ib, k, rb):
        pltpu.async_copy(table_hbm.at[idx_v.at[ib, 0].at[k]],
                         bufs[rb], gsems[rb])

    def wait_gather(rb):
        pltpu.make_async_copy(table_hbm.at[idx_v.at[0, 0].at[0]],
                              bufs[rb], gsems[rb]).wait()

    def add_off(ib):
        def step(i, c2):
            for j in range(CH // 16):
                sl = idx_v[ib, 0, i, pl.ds(j * 16, 16)]
                idx_v[ib, 0, i, pl.ds(j * 16, 16)] = sl + cN
            return c2
        lax.fori_loop(0, GC, step, 0)

    def wait_idx(ib):
        pltpu.make_async_copy(edges_hbm.at[s, 0], idx_v.at[ib], semi).wait()

    pltpu.sync_copy(edges_hbm.at[s, 0], idx_v.at[0])
    add_off(0)
    gather(0, 0, 0)
    pltpu.async_copy(edges_hbm.at[s, 1], idx_v.at[1], semi)

    for g in range(NG):
        b = g % 2
        if 1 <= g < NG - 1:
            pltpu.async_copy(edges_hbm.at[s, g + 1], idx_v.at[1 - b], semi)
        for k in range(GC):
            rb = k % 2
            if k < GC - 1:
                gather(b, k + 1, 1 - rb)
            elif g < NG - 1:
                gather(1 - b, 0, 1 - rb)
            if k == GC - 2 and g < NG - 1:
                wait_idx(1 - b)
                add_off(1 - b)
            wait_gather(rb)
            pltpu.sync_copy(bufs[rb], acc.at[idx_v.at[b, 1].at[k]], add=True)
    plsc.subcore_barrier()

    pltpu.sync_copy(acc.at[pl.ds(base, RPT)],
                    out_hbm.at[pl.ds(cN + base, RPT)])

    @pl.when(s == 0)
    def _():
        pltpu.sync_copy(acc.at[pl.ds(NS * RPT, 16)],
                        out_hbm.at[pl.ds(cN + NS * RPT, 16)])


def _edge_sums(table, edges):
    f = pl.kernel(
        _edge_sums_body,
        out_type=jax.ShapeDtypeStruct((2 * N, D), jnp.float32),
        mesh=plsc.VectorSubcoreMesh(core_axis_name="c", subcore_axis_name="s",
                                    num_cores=NC, num_subcores=NS),
        scratch_types=[
            pltpu.VMEM((2, 2, GC, CH), jnp.int32),
            pltpu.VMEM((2, CH, D), jnp.float32),
            pltpu.VMEM_SHARED((N + 16, D), jnp.float32),
            pltpu.SemaphoreType.DMA,
            pltpu.SemaphoreType.DMA,
            pltpu.SemaphoreType.DMA,
        ],
    )
    return f(table, edges)


NB = 10
R = N // NB


def _tables_body(t_ref, x_ref, tab_ref):
    m = jnp.maximum(x_ref[...], 0.0) + EPS
    e = jnp.exp(t_ref[0, 0] * m)
    tab_ref[0] = m * e
    tab_ref[1] = e


def _make_tables(x, t):
    return pl.pallas_call(
        _tables_body,
        grid=(NB,),
        in_specs=[
            pl.BlockSpec(memory_space=pltpu.SMEM),
            pl.BlockSpec((R, D), lambda i: (i, 0)),
        ],
        out_specs=pl.BlockSpec((2, R, D), lambda i: (0, i, 0)),
        out_shape=jax.ShapeDtypeStruct((2, N, D), jnp.float32),
    )(t.reshape(1, 1), x)


def _mlp1_body(sums_ref, x_ref, w1_ref, b1_ref, g_ref, be_ref,
               h_ref, ac_ref, acc_ref):
    i = pl.program_id(0)
    num = sums_ref[0]
    den = sums_ref[1]
    out = num / (den + 1e-16) + x_ref[...]
    h = jnp.dot(out, w1_ref[...], preferred_element_type=jnp.float32)
    h = h + b1_ref[0:1]
    h_ref[...] = h

    @pl.when(i == 0)
    def _():
        acc_ref[...] = jnp.zeros_like(acc_ref)

    acc_ref[0:1] += jnp.sum(h, axis=0, keepdims=True)
    acc_ref[1:2] += jnp.sum(h * h, axis=0, keepdims=True)

    @pl.when(i == NB - 1)
    def _():
        mu = acc_ref[0:1] / N
        var = acc_ref[1:2] / N - mu * mu
        a = g_ref[0:1] * lax.rsqrt(var + BN_EPS)
        ac_ref[0:1] = a
        ac_ref[1:2] = be_ref[0:1] - mu * a


def _mlp1(sums, x, w1, b1, g, be):
    return pl.pallas_call(
        _mlp1_body,
        grid=(NB,),
        in_specs=[
            pl.BlockSpec((2, R, D), lambda i: (0, i, 0)),
            pl.BlockSpec((R, D), lambda i: (i, 0)),
            pl.BlockSpec((D, DH), lambda i: (0, 0)),
            pl.BlockSpec((1, DH), lambda i: (0, 0)),
            pl.BlockSpec((1, DH), lambda i: (0, 0)),
            pl.BlockSpec((1, DH), lambda i: (0, 0)),
        ],
        out_specs=[
            pl.BlockSpec((R, DH), lambda i: (i, 0)),
            pl.BlockSpec((2, DH), lambda i: (0, 0)),
        ],
        out_shape=[
            jax.ShapeDtypeStruct((N, DH), jnp.float32),
            jax.ShapeDtypeStruct((2, DH), jnp.float32),
        ],
        scratch_shapes=[pltpu.VMEM((2, DH), jnp.float32)],
    )(sums, x, w1, b1.reshape(1, DH), g.reshape(1, DH), be.reshape(1, DH))


def _mlp2_tables_body(t_ref, h_ref, ac_ref, w2_ref, b2_ref, xn_ref, tab_ref):
    h = h_ref[...] * ac_ref[0:1] + ac_ref[1:2]
    r = jnp.maximum(h, 0.0)
    y = jnp.dot(r, w2_ref[...], preferred_element_type=jnp.float32)
    xn = jnp.maximum(y + b2_ref[0:1], 0.0)
    xn_ref[...] = xn
    m = xn + EPS
    e = jnp.exp(t_ref[0, 0] * m)
    tab_ref[0] = m * e
    tab_ref[1] = e


def _mlp2_tables(h, ac, w2, b2, t_next):
    return pl.pallas_call(
        _mlp2_tables_body,
        grid=(NB,),
        in_specs=[
            pl.BlockSpec(memory_space=pltpu.SMEM),
            pl.BlockSpec((R, DH), lambda i: (i, 0)),
            pl.BlockSpec((2, DH), lambda i: (0, 0)),
            pl.BlockSpec((DH, D), lambda i: (0, 0)),
            pl.BlockSpec((1, D), lambda i: (0, 0)),
        ],
        out_specs=[
            pl.BlockSpec((R, D), lambda i: (i, 0)),
            pl.BlockSpec((2, R, D), lambda i: (0, i, 0)),
        ],
        out_shape=[
            jax.ShapeDtypeStruct((N, D), jnp.float32),
            jax.ShapeDtypeStruct((2, N, D), jnp.float32),
        ],
    )(t_next.reshape(1, 1), h, ac, w2, b2.reshape(1, D))


def _mlp2_final_body(h_ref, ac_ref, w2_ref, b2_ref, xn_ref):
    h = h_ref[...] * ac_ref[0:1] + ac_ref[1:2]
    r = jnp.maximum(h, 0.0)
    y = jnp.dot(r, w2_ref[...], preferred_element_type=jnp.float32)
    xn_ref[...] = jnp.maximum(y + b2_ref[0:1], 0.0)


def _mlp2_final(h, ac, w2, b2):
    return pl.pallas_call(
        _mlp2_final_body,
        grid=(NB,),
        in_specs=[
            pl.BlockSpec((R, DH), lambda i: (i, 0)),
            pl.BlockSpec((2, DH), lambda i: (0, 0)),
            pl.BlockSpec((DH, D), lambda i: (0, 0)),
            pl.BlockSpec((1, D), lambda i: (0, 0)),
        ],
        out_specs=pl.BlockSpec((R, D), lambda i: (i, 0)),
        out_shape=jax.ShapeDtypeStruct((N, D), jnp.float32),
    )(h, ac, w2, b2.reshape(1, D))


def kernel(x, edge_index, t0, W1_0, b1_0, g_0, be_0, W2_0, b2_0,
           t1, W1_1, b1_1, g_1, be_1, W2_1, b2_1):
    npad = EP - E
    e32 = edge_index.astype(jnp.int32)
    pad = jnp.stack([jnp.zeros((npad,), jnp.int32),
                     jnp.full((npad,), N, jnp.int32)])
    edges = jnp.concatenate([e32, pad], axis=1)
    edges = edges.reshape(2, NS, NG, GC, CH).transpose(1, 2, 0, 3, 4)

    tab0 = _make_tables(x, t0).reshape(2 * N, D)
    sums0 = _edge_sums(tab0, edges).reshape(2, N, D)
    h0, ac0 = _mlp1(sums0, x, W1_0, b1_0, g_0, be_0)
    x1, tab1 = _mlp2_tables(h0, ac0, W2_0, b2_0, t1)

    sums1 = _edge_sums(tab1.reshape(2 * N, D), edges).reshape(2, N, D)
    h1, ac1 = _mlp1(sums1, x1, W1_1, b1_1, g_1, be_1)
    return _mlp2_final(h1, ac1, W2_1, b2_1)


# speedup vs baseline: 2.2391x; 2.2391x over previous
import functools

import jax
import jax.numpy as jnp
from jax import lax
from jax.experimental import pallas as pl
from jax.experimental.pallas import tpu as pltpu
from jax.experimental.pallas import tpu_sc as plsc

N = 10000
E = 320000
D = 128
DH = 256
EPS = 1e-7
BN_EPS = 1e-5

NC = 2
NS = 16
CH = 80
EPT = E // NS
NK = EPT // CH
GC = 10
NG = NK // GC
RPT = 624


def _edge_sums_body(table_hbm, edges_hbm, out_hbm,
                    idx_v, rows_v, acc, semg0, semg1, semi):
    c = lax.axis_index("c")
    s = lax.axis_index("s")
    cN = (c * N).astype(jnp.int32)
    buf0 = rows_v.at[0]
    buf1 = rows_v.at[1]

    zvec = jnp.zeros((16,), jnp.float32)

    def zero_row(i, carry):
        for j in range(D // 16):
            rows_v[0, i, pl.ds(j * 16, 16)] = zvec
        return carry

    lax.fori_loop(0, CH, zero_row, 0)
    base = s * RPT
    for r in range(RPT // CH):
        pltpu.sync_copy(buf0, acc.at[pl.ds(base + r * CH, CH)])
    rem = RPT - (RPT // CH) * CH
    pltpu.sync_copy(buf0.at[pl.ds(0, rem)],
                    acc.at[pl.ds(base + (RPT // CH) * CH, rem)])

    @pl.when(s == 0)
    def _():
        pltpu.sync_copy(buf0.at[pl.ds(0, 16)], acc.at[pl.ds(NS * RPT, 16)])

    plsc.subcore_barrier()

    bufs = (buf0, buf1)
    gsems = (semg0, semg1)

    def gather(ib, k, rb):
        pltpu.async_copy(table_hbm.at[idx_v.at[ib, 0].at[k]],
                         bufs[rb], gsems[rb])

    def wait_gather(rb):
        pltpu.make_async_copy(table_hbm.at[idx_v.at[0, 0].at[0]],
                              bufs[rb], gsems[rb]).wait()

    def add_off(ib):
        def step(i, c2):
            for j in range(CH // 16):
                sl = idx_v[ib, 0, i, pl.ds(j * 16, 16)]
                idx_v[ib, 0, i, pl.ds(j * 16, 16)] = sl + cN
            return c2
        lax.fori_loop(0, GC, step, 0)

    def wait_idx(ib):
        pltpu.make_async_copy(edges_hbm.at[s, 0], idx_v.at[ib], semi).wait()

    pltpu.sync_copy(edges_hbm.at[s, 0], idx_v.at[0])
    add_off(0)
    gather(0, 0, 0)
    pltpu.async_copy(edges_hbm.at[s, 1], idx_v.at[1], semi)

    for g in range(NG):
        b = g % 2
        if 1 <= g < NG - 1:
            pltpu.async_copy(edges_hbm.at[s, g + 1], idx_v.at[1 - b], semi)
        for k in range(GC):
            rb = k % 2
            if k < GC - 1:
                gather(b, k + 1, 1 - rb)
            elif g < NG - 1:
                gather(1 - b, 0, 1 - rb)
            if k == GC - 2 and g < NG - 1:
                wait_idx(1 - b)
                add_off(1 - b)
            wait_gather(rb)
            pltpu.sync_copy(bufs[rb], acc.at[idx_v.at[b, 1].at[k]], add=True)
    plsc.subcore_barrier()

    pltpu.sync_copy(acc.at[pl.ds(base, RPT)],
                    out_hbm.at[pl.ds(cN + base, RPT)])

    @pl.when(s == 0)
    def _():
        pltpu.sync_copy(acc.at[pl.ds(NS * RPT, 16)],
                        out_hbm.at[pl.ds(cN + NS * RPT, 16)])


def _edge_sums(table, edges):
    f = pl.kernel(
        _edge_sums_body,
        out_type=jax.ShapeDtypeStruct((2 * N, D), jnp.float32),
        mesh=plsc.VectorSubcoreMesh(core_axis_name="c", subcore_axis_name="s",
                                    num_cores=NC, num_subcores=NS),
        scratch_types=[
            pltpu.VMEM((2, 2, GC, CH), jnp.int32),
            pltpu.VMEM((2, CH, D), jnp.float32),
            pltpu.VMEM_SHARED((N, D), jnp.float32),
            pltpu.SemaphoreType.DMA,
            pltpu.SemaphoreType.DMA,
            pltpu.SemaphoreType.DMA,
        ],
    )
    return f(table, edges)


NB = 10
R = N // NB


def _tables_body(t_ref, x_ref, tab_ref):
    m = jnp.maximum(x_ref[...], 0.0) + EPS
    e = jnp.exp(t_ref[0, 0] * m)
    tab_ref[0] = m * e
    tab_ref[1] = e


def _make_tables(x, t):
    return pl.pallas_call(
        _tables_body,
        grid=(NB,),
        in_specs=[
            pl.BlockSpec(memory_space=pltpu.SMEM),
            pl.BlockSpec((R, D), lambda i: (i, 0)),
        ],
        out_specs=pl.BlockSpec((2, R, D), lambda i: (0, i, 0)),
        out_shape=jax.ShapeDtypeStruct((2, N, D), jnp.float32),
    )(t.reshape(1, 1), x)


def _block_core(p, i, sums_ref, x_ref, w1_ref, b1_ref, g_ref, be_ref,
                w2_ref, b2_ref, xn_ref, hs_ref, st_ref, ac_ref, emit):
    @pl.when(p == 0)
    def _():
        out = sums_ref[0] / (sums_ref[1] + 1e-16) + x_ref[...]
        h = jnp.dot(out, w1_ref[...], preferred_element_type=jnp.float32)
        h = h + b1_ref[0:1]
        hs_ref[pl.ds(i * R, R), :] = h

        @pl.when(i == 0)
        def _():
            st_ref[...] = jnp.zeros_like(st_ref)

        st_ref[0:1] += jnp.sum(h, axis=0, keepdims=True)
        st_ref[1:2] += jnp.sum(h * h, axis=0, keepdims=True)

        @pl.when(i == NB - 1)
        def _():
            mu = st_ref[0:1] / N
            var = st_ref[1:2] / N - mu * mu
            a = g_ref[0:1] * lax.rsqrt(var + BN_EPS)
            ac_ref[0:1] = a
            ac_ref[1:2] = be_ref[0:1] - mu * a

    @pl.when(p == 1)
    def _():
        h = hs_ref[pl.ds(i * R, R), :]
        r = jnp.maximum(h * ac_ref[0:1] + ac_ref[1:2], 0.0)
        y = jnp.dot(r, w2_ref[...], preferred_element_type=jnp.float32)
        xn = jnp.maximum(y + b2_ref[0:1], 0.0)
        xn_ref[...] = xn
        emit(xn)


def _block_tables_body(t_ref, sums_ref, x_ref, w1_ref, b1_ref, g_ref,
                       be_ref, w2_ref, b2_ref, xn_ref, tab_ref,
                       hs_ref, st_ref, ac_ref):
    def emit(xn):
        m = xn + EPS
        e = jnp.exp(t_ref[0, 0] * m)
        tab_ref[0] = m * e
        tab_ref[1] = e

    _block_core(pl.program_id(0), pl.program_id(1), sums_ref, x_ref,
                w1_ref, b1_ref, g_ref, be_ref, w2_ref, b2_ref,
                xn_ref, hs_ref, st_ref, ac_ref, emit)


def _block_final_body(sums_ref, x_ref, w1_ref, b1_ref, g_ref, be_ref,
                      w2_ref, b2_ref, xn_ref, hs_ref, st_ref, ac_ref):
    _block_core(pl.program_id(0), pl.program_id(1), sums_ref, x_ref,
                w1_ref, b1_ref, g_ref, be_ref, w2_ref, b2_ref,
                xn_ref, hs_ref, st_ref, ac_ref, lambda xn: None)


def _block_in_specs():
    return [
        pl.BlockSpec((2, R, D), lambda p, i: (0, jnp.where(p == 0, i, 0), 0)),
        pl.BlockSpec((R, D), lambda p, i: (jnp.where(p == 0, i, 0), 0)),
        pl.BlockSpec((D, DH), lambda p, i: (0, 0)),
        pl.BlockSpec((1, DH), lambda p, i: (0, 0)),
        pl.BlockSpec((1, DH), lambda p, i: (0, 0)),
        pl.BlockSpec((1, DH), lambda p, i: (0, 0)),
        pl.BlockSpec((DH, D), lambda p, i: (0, 0)),
        pl.BlockSpec((1, D), lambda p, i: (0, 0)),
    ]


_BLOCK_SCRATCH = [
    pltpu.VMEM((N, DH), jnp.float32),
    pltpu.VMEM((2, DH), jnp.float32),
    pltpu.VMEM((2, DH), jnp.float32),
]


def _block_tables(sums, x, w1, b1, g, be, w2, b2, t_next):
    return pl.pallas_call(
        _block_tables_body,
        grid=(2, NB),
        in_specs=[pl.BlockSpec(memory_space=pltpu.SMEM)] + _block_in_specs(),
        out_specs=[
            pl.BlockSpec((R, D), lambda p, i: (i, 0)),
            pl.BlockSpec((2, R, D), lambda p, i: (0, i, 0)),
        ],
        out_shape=[
            jax.ShapeDtypeStruct((N, D), jnp.float32),
            jax.ShapeDtypeStruct((2, N, D), jnp.float32),
        ],
        scratch_shapes=_BLOCK_SCRATCH,
    )(t_next.reshape(1, 1), sums, x, w1, b1.reshape(1, DH),
      g.reshape(1, DH), be.reshape(1, DH), w2, b2.reshape(1, D))


def _block_final(sums, x, w1, b1, g, be, w2, b2):
    return pl.pallas_call(
        _block_final_body,
        grid=(2, NB),
        in_specs=_block_in_specs(),
        out_specs=pl.BlockSpec((R, D), lambda p, i: (i, 0)),
        out_shape=jax.ShapeDtypeStruct((N, D), jnp.float32),
        scratch_shapes=_BLOCK_SCRATCH,
    )(sums, x, w1, b1.reshape(1, DH), g.reshape(1, DH), be.reshape(1, DH),
      w2, b2.reshape(1, D))


def kernel(x, edge_index, t0, W1_0, b1_0, g_0, be_0, W2_0, b2_0,
           t1, W1_1, b1_1, g_1, be_1, W2_1, b2_1):
    edges = edge_index.astype(jnp.int32).reshape(2, NS, NG, GC, CH)
    edges = edges.transpose(1, 2, 0, 3, 4)

    tab0 = _make_tables(x, t0).reshape(2 * N, D)
    sums0 = _edge_sums(tab0, edges).reshape(2, N, D)
    x1, tab1 = _block_tables(sums0, x, W1_0, b1_0, g_0, be_0, W2_0, b2_0, t1)

    sums1 = _edge_sums(tab1.reshape(2 * N, D), edges).reshape(2, N, D)
    return _block_final(sums1, x1, W1_1, b1_1, g_1, be_1, W2_1, b2_1)


# speedup vs baseline: 2.2537x; 1.0065x over previous
import functools

import jax
import jax.numpy as jnp
from jax import lax
from jax.experimental import pallas as pl
from jax.experimental.pallas import tpu as pltpu
from jax.experimental.pallas import tpu_sc as plsc

N = 10000
E = 320000
D = 128
DH = 256
EPS = 1e-7
BN_EPS = 1e-5

NC = 2
NS = 16
CH = 80
EPT = E // NS
NK = EPT // CH
GC = 10
NG = NK // GC
RPT = 624


def _edge_sums_body(table_hbm, edges_hbm, out_hbm,
                    idx_v, rows_v, acc, semg0, semg1, semi):
    c = lax.axis_index("c")
    s = lax.axis_index("s")
    cN = (c * N).astype(jnp.int32)
    buf0 = rows_v.at[0]
    buf1 = rows_v.at[1]

    zvec = jnp.zeros((16,), jnp.float32)

    def zero_row(i, carry):
        for j in range(D // 16):
            rows_v[0, i, pl.ds(j * 16, 16)] = zvec
        return carry

    lax.fori_loop(0, CH, zero_row, 0)
    base = s * RPT
    for r in range(RPT // CH):
        pltpu.sync_copy(buf0, acc.at[pl.ds(base + r * CH, CH)])
    rem = RPT - (RPT // CH) * CH
    pltpu.sync_copy(buf0.at[pl.ds(0, rem)],
                    acc.at[pl.ds(base + (RPT // CH) * CH, rem)])

    @pl.when(s == 0)
    def _():
        pltpu.sync_copy(buf0.at[pl.ds(0, 16)], acc.at[pl.ds(NS * RPT, 16)])

    plsc.subcore_barrier()

    bufs = (buf0, buf1)
    gsems = (semg0, semg1)

    tab_c = table_hbm.at[c]

    def gather(ib, k, rb):
        pltpu.async_copy(tab_c.at[idx_v.at[ib, 0].at[k]],
                         bufs[rb], gsems[rb])

    def wait_gather(rb):
        pltpu.make_async_copy(tab_c.at[idx_v.at[0, 0].at[0]],
                              bufs[rb], gsems[rb]).wait()

    def wait_idx(ib):
        pltpu.make_async_copy(edges_hbm.at[s, 0], idx_v.at[ib], semi).wait()

    pltpu.sync_copy(edges_hbm.at[s, 0], idx_v.at[0])
    gather(0, 0, 0)
    pltpu.async_copy(edges_hbm.at[s, 1], idx_v.at[1], semi)

    for g in range(NG):
        b = g % 2
        if 1 <= g < NG - 1:
            pltpu.async_copy(edges_hbm.at[s, g + 1], idx_v.at[1 - b], semi)
        for k in range(GC):
            rb = k % 2
            if k < GC - 1:
                gather(b, k + 1, 1 - rb)
            elif g < NG - 1:
                gather(1 - b, 0, 1 - rb)
            if k == GC - 2 and g < NG - 1:
                wait_idx(1 - b)
            wait_gather(rb)
            pltpu.sync_copy(bufs[rb], acc.at[idx_v.at[b, 1].at[k]], add=True)
    plsc.subcore_barrier()

    pltpu.sync_copy(acc.at[pl.ds(base, RPT)],
                    out_hbm.at[pl.ds(cN + base, RPT)])

    @pl.when(s == 0)
    def _():
        pltpu.sync_copy(acc.at[pl.ds(NS * RPT, 16)],
                        out_hbm.at[pl.ds(cN + NS * RPT, 16)])


def _edge_sums(table, edges):
    f = pl.kernel(
        _edge_sums_body,
        out_type=jax.ShapeDtypeStruct((2 * N, D), jnp.float32),
        mesh=plsc.VectorSubcoreMesh(core_axis_name="c", subcore_axis_name="s",
                                    num_cores=NC, num_subcores=NS),
        scratch_types=[
            pltpu.VMEM((2, 2, GC, CH), jnp.int32),
            pltpu.VMEM((2, CH, D), jnp.float32),
            pltpu.VMEM_SHARED((N, D), jnp.float32),
            pltpu.SemaphoreType.DMA,
            pltpu.SemaphoreType.DMA,
            pltpu.SemaphoreType.DMA,
        ],
    )
    return f(table, edges)


NB = 10
R = N // NB


def _tables_body(t_ref, x_ref, tab_ref):
    m = jnp.maximum(x_ref[...], 0.0) + EPS
    e = jnp.exp(t_ref[0, 0] * m)
    tab_ref[0] = m * e
    tab_ref[1] = e


def _make_tables(x, t):
    return pl.pallas_call(
        _tables_body,
        grid=(NB,),
        in_specs=[
            pl.BlockSpec(memory_space=pltpu.SMEM),
            pl.BlockSpec((R, D), lambda i: (i, 0)),
        ],
        out_specs=pl.BlockSpec((2, R, D), lambda i: (0, i, 0)),
        out_shape=jax.ShapeDtypeStruct((2, N, D), jnp.float32),
    )(t.reshape(1, 1), x)


def _block_core(p, i, sums_ref, x_ref, w1_ref, b1_ref, g_ref, be_ref,
                w2_ref, b2_ref, xn_ref, hs_ref, st_ref, ac_ref, emit):
    @pl.when(p == 0)
    def _():
        out = sums_ref[0] / (sums_ref[1] + 1e-16) + x_ref[...]
        h = jnp.dot(out, w1_ref[...], preferred_element_type=jnp.float32)
        h = h + b1_ref[0:1]
        hs_ref[pl.ds(i * R, R), :] = h

        @pl.when(i == 0)
        def _():
            st_ref[...] = jnp.zeros_like(st_ref)

        st_ref[0:1] += jnp.sum(h, axis=0, keepdims=True)
        st_ref[1:2] += jnp.sum(h * h, axis=0, keepdims=True)

        @pl.when(i == NB - 1)
        def _():
            mu = st_ref[0:1] / N
            var = st_ref[1:2] / N - mu * mu
            a = g_ref[0:1] * lax.rsqrt(var + BN_EPS)
            ac_ref[0:1] = a
            ac_ref[1:2] = be_ref[0:1] - mu * a

    @pl.when(p == 1)
    def _():
        h = hs_ref[pl.ds(i * R, R), :]
        r = jnp.maximum(h * ac_ref[0:1] + ac_ref[1:2], 0.0)
        y = jnp.dot(r, w2_ref[...], preferred_element_type=jnp.float32)
        xn = jnp.maximum(y + b2_ref[0:1], 0.0)
        xn_ref[...] = xn
        emit(xn)


def _block_tables_body(t_ref, sums_ref, x_ref, w1_ref, b1_ref, g_ref,
                       be_ref, w2_ref, b2_ref, xn_ref, tab_ref,
                       hs_ref, st_ref, ac_ref):
    def emit(xn):
        m = xn + EPS
        e = jnp.exp(t_ref[0, 0] * m)
        tab_ref[0] = m * e
        tab_ref[1] = e

    _block_core(pl.program_id(0), pl.program_id(1), sums_ref, x_ref,
                w1_ref, b1_ref, g_ref, be_ref, w2_ref, b2_ref,
                xn_ref, hs_ref, st_ref, ac_ref, emit)


def _block_final_body(sums_ref, x_ref, w1_ref, b1_ref, g_ref, be_ref,
                      w2_ref, b2_ref, xn_ref, hs_ref, st_ref, ac_ref):
    _block_core(pl.program_id(0), pl.program_id(1), sums_ref, x_ref,
                w1_ref, b1_ref, g_ref, be_ref, w2_ref, b2_ref,
                xn_ref, hs_ref, st_ref, ac_ref, lambda xn: None)


def _block_in_specs():
    return [
        pl.BlockSpec((2, R, D), lambda p, i: (0, jnp.where(p == 0, i, 0), 0)),
        pl.BlockSpec((R, D), lambda p, i: (jnp.where(p == 0, i, 0), 0)),
        pl.BlockSpec((D, DH), lambda p, i: (0, 0)),
        pl.BlockSpec((1, DH), lambda p, i: (0, 0)),
        pl.BlockSpec((1, DH), lambda p, i: (0, 0)),
        pl.BlockSpec((1, DH), lambda p, i: (0, 0)),
        pl.BlockSpec((DH, D), lambda p, i: (0, 0)),
        pl.BlockSpec((1, D), lambda p, i: (0, 0)),
    ]


_BLOCK_SCRATCH = [
    pltpu.VMEM((N, DH), jnp.float32),
    pltpu.VMEM((2, DH), jnp.float32),
    pltpu.VMEM((2, DH), jnp.float32),
]


def _block_tables(sums, x, w1, b1, g, be, w2, b2, t_next):
    return pl.pallas_call(
        _block_tables_body,
        grid=(2, NB),
        in_specs=[pl.BlockSpec(memory_space=pltpu.SMEM)] + _block_in_specs(),
        out_specs=[
            pl.BlockSpec((R, D), lambda p, i: (i, 0)),
            pl.BlockSpec((2, R, D), lambda p, i: (0, i, 0)),
        ],
        out_shape=[
            jax.ShapeDtypeStruct((N, D), jnp.float32),
            jax.ShapeDtypeStruct((2, N, D), jnp.float32),
        ],
        scratch_shapes=_BLOCK_SCRATCH,
    )(t_next.reshape(1, 1), sums, x, w1, b1.reshape(1, DH),
      g.reshape(1, DH), be.reshape(1, DH), w2, b2.reshape(1, D))


def _block_final(sums, x, w1, b1, g, be, w2, b2):
    return pl.pallas_call(
        _block_final_body,
        grid=(2, NB),
        in_specs=_block_in_specs(),
        out_specs=pl.BlockSpec((R, D), lambda p, i: (i, 0)),
        out_shape=jax.ShapeDtypeStruct((N, D), jnp.float32),
        scratch_shapes=_BLOCK_SCRATCH,
    )(sums, x, w1, b1.reshape(1, DH), g.reshape(1, DH), be.reshape(1, DH),
      w2, b2.reshape(1, D))


def kernel(x, edge_index, t0, W1_0, b1_0, g_0, be_0, W2_0, b2_0,
           t1, W1_1, b1_1, g_1, be_1, W2_1, b2_1):
    edges = edge_index.astype(jnp.int32).reshape(2, NS, NG, GC, CH)
    edges = edges.transpose(1, 2, 0, 3, 4)

    tab0 = _make_tables(x, t0)
    sums0 = _edge_sums(tab0, edges).reshape(2, N, D)
    x1, tab1 = _block_tables(sums0, x, W1_0, b1_0, g_0, be_0, W2_0, b2_0, t1)

    sums1 = _edge_sums(tab1, edges).reshape(2, N, D)
    return _block_final(sums1, x1, W1_1, b1_1, g_1, be_1, W2_1, b2_1)
